# baseline (speedup 1.0000x reference)
import functools

import jax
import jax.numpy as jnp
from jax import lax
from jax.experimental import pallas as pl
from jax.experimental.pallas import tpu as pltpu
from jax.experimental.pallas import tpu_sc as plsc

NV = 10000
NPAD = 10240
NB = 512
NG = NPAD // NB
E = 160000
D = 128
K = 5000
KPAD = 5120
NW = 32
WROWS = 8
NPASS = NPAD // (NW * WROWS)
ECH = 4000

_MESH = plsc.VectorSubcoreMesh(core_axis_name="c", subcore_axis_name="s")


def _wid():
    return lax.axis_index("s") * 2 + lax.axis_index("c")


@functools.partial(
    pl.kernel,
    out_type=jax.ShapeDtypeStruct((NPAD * NPAD,), jnp.float32),
    mesh=_MESH,
    compiler_params=pltpu.CompilerParams(needs_layout_passes=False),
    scratch_types=[
        pltpu.VMEM((WROWS * NPAD,), jnp.float32),
        pltpu.VMEM((ECH,), jnp.int32),
        pltpu.VMEM((ECH,), jnp.int32),
    ],
)
def _k1_build(dst_hbm, src_hbm, out_hbm, win, dbuf, sbuf):
    w = _wid()
    zero16 = jnp.zeros((16,), jnp.float32)
    one16 = jnp.ones((16,), jnp.float32)
    lane = lax.iota(jnp.int32, 16)

    def one_pass(p, _):
        base = (p * NW + w) * WROWS

        def zcol(j, _):
            win[pl.ds(j * 16, 16)] = zero16
            return 0
        lax.fori_loop(0, WROWS * NPAD // 16, zcol, 0)

        for r in range(WROWS):
            g = base + r
            @pl.when(g < NV)
            def _():
                plsc.addupdate_scatter(
                    win, [jnp.full((16,), r * NPAD + g, jnp.int32)],
                    one16, mask=lane == 0)

        def chunk(c, _):
            pltpu.sync_copy(dst_hbm.at[pl.ds(c * ECH, ECH)], dbuf)
            pltpu.sync_copy(src_hbm.at[pl.ds(c * ECH, ECH)], sbuf)

            def grp(g, _):
                d = dbuf[pl.ds(g * 16, 16)]
                s = sbuf[pl.ds(g * 16, 16)]
                rel = d - base
                m = (rel >= 0) & (rel < WROWS)
                flat = jnp.where(m, rel * NPAD + s, 0)
                plsc.addupdate_scatter(win, [flat], one16, mask=m)
                return 0
            lax.fori_loop(0, ECH // 16, grp, 0)
            return 0
        lax.fori_loop(0, E // ECH, chunk, 0)

        pltpu.sync_copy(win, out_hbm.at[pl.ds(base * NPAD, WROWS * NPAD)])
        return 0

    lax.fori_loop(0, NPASS, one_pass, 0)


def _cast_kernel(x_ref, o_ref):
    o_ref[...] = x_ref[...].astype(jnp.bfloat16)


def _k15_cast(bd):
    return pl.pallas_call(
        _cast_kernel,
        grid=(NG, NG),
        in_specs=[pl.BlockSpec((NB, NB), lambda i, j: (i, j))],
        out_specs=pl.BlockSpec((NB, NB), lambda i, j: (i, j)),
        out_shape=jax.ShapeDtypeStruct((NPAD, NPAD), jnp.bfloat16),
    )(bd)


def _mm_kernel(a_ref, b_ref, c_ref, acc_ref):
    k = pl.program_id(2)

    @pl.when(k == 0)
    def _():
        acc_ref[...] = jnp.zeros_like(acc_ref)

    acc_ref[...] += jnp.dot(a_ref[...], b_ref[...],
                            preferred_element_type=jnp.float32)

    @pl.when(k == NG - 1)
    def _():
        i = pl.program_id(0)
        j = pl.program_id(1)
        acc = acc_ref[...]
        rows = lax.broadcasted_iota(jnp.int32, (NB, NB), 0)
        cols = lax.broadcasted_iota(jnp.int32, (NB, NB), 1)
        isdiag = jnp.logical_and(rows == cols, i == j)
        c_ref[...] = jnp.where(isdiag, 1.0, acc).astype(jnp.bfloat16)


def _k2_square(bb):
    return pl.pallas_call(
        _mm_kernel,
        grid=(NG, NG, NG),
        in_specs=[
            pl.BlockSpec((NB, NB), lambda i, j, k: (i, k)),
            pl.BlockSpec((NB, NB), lambda i, j, k: (k, j)),
        ],
        out_specs=pl.BlockSpec((NB, NB), lambda i, j, k: (i, j)),
        out_shape=jax.ShapeDtypeStruct((NPAD, NPAD), jnp.bfloat16),
        scratch_shapes=[pltpu.VMEM((NB, NB), jnp.float32)],
        compiler_params=pltpu.CompilerParams(
            dimension_semantics=("parallel", "parallel", "arbitrary")),
    )(bb, bb)


def _dc_kernel(a_ref, b_ref, o_ref):
    j = pl.program_id(1)
    p = jnp.dot(a_ref[...], b_ref[...], preferred_element_type=jnp.float32)
    rows = lax.broadcasted_iota(jnp.int32, (NB, NB), 0)
    cols = lax.broadcasted_iota(jnp.int32, (NB, NB), 1)
    s = jnp.sum(jnp.where(rows == cols, p, 0.0), axis=1)

    @pl.when(j == 0)
    def _():
        o_ref[...] = s

    @pl.when(j != 0)
    def _():
        o_ref[...] = o_ref[...] + s


def _k3_dc(cb):
    return pl.pallas_call(
        _dc_kernel,
        grid=(NG, NG),
        in_specs=[
            pl.BlockSpec((NB, NB), lambda i, j: (i, j)),
            pl.BlockSpec((NB, NB), lambda i, j: (j, i)),
        ],
        out_specs=pl.BlockSpec((NB,), lambda i, j: (i,)),
        out_shape=jax.ShapeDtypeStruct((NPAD,), jnp.float32),
        compiler_params=pltpu.CompilerParams(
            dimension_semantics=("parallel", "arbitrary")),
    )(cb, cb)


def _capp_kernel(c_ref, z_ref, o_ref):
    j = pl.program_id(1)
    z = z_ref[...]
    zh = z.astype(jnp.bfloat16)
    zl = (z - zh.astype(jnp.float32)).astype(jnp.bfloat16)
    c = c_ref[...]
    p = (jnp.dot(c, zh, preferred_element_type=jnp.float32)
         + jnp.dot(c, zl, preferred_element_type=jnp.float32))

    @pl.when(j == 0)
    def _():
        o_ref[...] = p

    @pl.when(j != 0)
    def _():
        o_ref[...] = o_ref[...] + p


def _capp(cb, z):
    return pl.pallas_call(
        _capp_kernel,
        grid=(NG, NG),
        in_specs=[
            pl.BlockSpec((NB, NB), lambda i, j: (i, j)),
            pl.BlockSpec((NB, D), lambda i, j: (j, 0)),
        ],
        out_specs=pl.BlockSpec((NB, D), lambda i, j: (i, 0)),
        out_shape=jax.ShapeDtypeStruct((NPAD, D), jnp.float32),
        compiler_params=pltpu.CompilerParams(
            dimension_semantics=("parallel", "arbitrary")),
    )(cb, z)


MB = 1024


def _t3_kernel(h_ref, v_ref, dinv_ref, w_ref, o_ref):
    xp = h_ref[...] * v_ref[...][:, None]
    y = jnp.dot(xp, w_ref[...], preferred_element_type=jnp.float32)
    o_ref[...] = dinv_ref[...][:, None] * y


def _t3(hpad, vfull, dinv2f, w1):
    return pl.pallas_call(
        _t3_kernel,
        grid=(NPAD // MB,),
        in_specs=[
            pl.BlockSpec((MB, D), lambda m: (m, 0)),
            pl.BlockSpec((MB,), lambda m: (m,)),
            pl.BlockSpec((MB,), lambda m: (m,)),
            pl.BlockSpec((D, D), lambda m: (0, 0)),
        ],
        out_specs=pl.BlockSpec((MB, D), lambda m: (m, 0)),
        out_shape=jax.ShapeDtypeStruct((NPAD, D), jnp.float32),
    )(hpad, vfull, dinv2f, w1)


def _t4_kernel(u_ref, z_ref, dinv_ref, dc_ref, s_ref, w_ref, b_ref, o_ref):
    z = z_ref[...]
    dinv = dinv_ref[...][:, None]
    ah_z = u_ref[...] - dc_ref[...][:, None] * z + 2.0 * z
    h2 = jnp.maximum(dinv * ah_z + b_ref[...][None, :], 0.0)
    h2 = h2 * s_ref[...][:, None]
    y = jnp.dot(h2, w_ref[...], preferred_element_type=jnp.float32)
    o_ref[...] = dinv * y


def _t4(u2, z2, dinv2f, dcf, s_ind, w2, b1):
    return pl.pallas_call(
        _t4_kernel,
        grid=(NPAD // MB,),
        in_specs=[
            pl.BlockSpec((MB, D), lambda m: (m, 0)),
            pl.BlockSpec((MB, D), lambda m: (m, 0)),
            pl.BlockSpec((MB,), lambda m: (m,)),
            pl.BlockSpec((MB,), lambda m: (m,)),
            pl.BlockSpec((MB,), lambda m: (m,)),
            pl.BlockSpec((D, D), lambda m: (0, 0)),
            pl.BlockSpec((D,), lambda m: (0,)),
        ],
        out_specs=pl.BlockSpec((MB, D), lambda m: (m, 0)),
        out_shape=jax.ShapeDtypeStruct((NPAD, D), jnp.float32),
    )(u2, z2, dinv2f, dcf, s_ind, w2, b1)


def _t5_kernel(u_ref, z_ref, dinv_ref, dc_ref, b_ref, o_ref):
    z = z_ref[...]
    ah_z = u_ref[...] - dc_ref[...][:, None] * z + 2.0 * z
    o_ref[...] = dinv_ref[...][:, None] * ah_z + b_ref[...][None, :]


def _t5(u4, z3, dinv2f, dcf, b2):
    return pl.pallas_call(
        _t5_kernel,
        grid=(NPAD // MB,),
        in_specs=[
            pl.BlockSpec((MB, D), lambda m: (m, 0)),
            pl.BlockSpec((MB, D), lambda m: (m, 0)),
            pl.BlockSpec((MB,), lambda m: (m,)),
            pl.BlockSpec((MB,), lambda m: (m,)),
            pl.BlockSpec((D,), lambda m: (0,)),
        ],
        out_specs=pl.BlockSpec((MB, D), lambda m: (m, 0)),
        out_shape=jax.ShapeDtypeStruct((NPAD, D), jnp.float32),
    )(u4, z3, dinv2f, dcf, b2)


@functools.partial(
    pl.kernel,
    out_type=(jax.ShapeDtypeStruct((NPAD,), jnp.float32),
              jax.ShapeDtypeStruct((NPAD,), jnp.float32)),
    mesh=_MESH,
    compiler_params=pltpu.CompilerParams(needs_layout_passes=False),
    scratch_types=[
        pltpu.VMEM((KPAD,), jnp.int32),
        pltpu.VMEM((KPAD,), jnp.float32),
        pltpu.VMEM((NPAD,), jnp.float32),
        pltpu.VMEM((NPAD,), jnp.float32),
    ],
)
def _k55_scatter(perm_hbm, vals_hbm, sind_hbm, vfull_hbm, pbuf, vbuf, sb, vb):
    w = _wid()

    @pl.when(w == 0)
    def _():
        zero16 = jnp.zeros((16,), jnp.float32)
        one16 = jnp.ones((16,), jnp.float32)
        pltpu.sync_copy(perm_hbm, pbuf)
        pltpu.sync_copy(vals_hbm, vbuf)

        def z(j, _):
            sb[pl.ds(j * 16, 16)] = zero16
            vb[pl.ds(j * 16, 16)] = zero16
            return 0
        lax.fori_loop(0, NPAD // 16, z, 0)

        def g(i, _):
            p16 = pbuf[pl.ds(i * 16, 16)]
            v16 = vbuf[pl.ds(i * 16, 16)]
            plsc.store_scatter(sb, [p16], one16)
            plsc.store_scatter(vb, [p16], v16)
            return 0
        lax.fori_loop(0, KPAD // 16, g, 0)

        pltpu.sync_copy(sb, sind_hbm)
        pltpu.sync_copy(vb, vfull_hbm)


GW = KPAD // NW


@functools.partial(
    pl.kernel,
    out_type=jax.ShapeDtypeStruct((KPAD, D), jnp.float32),
    mesh=_MESH,
    compiler_params=pltpu.CompilerParams(needs_layout_passes=False),
    scratch_types=[
        pltpu.VMEM((GW,), jnp.int32),
        pltpu.VMEM((GW, D), jnp.float32),
        pltpu.SemaphoreType.DMA,
    ],
)
def _k7_gather(src_hbm, perm_hbm, out_hbm, idxv, rows, sem):
    w = _wid()
    base = w * GW
    pltpu.sync_copy(perm_hbm.at[pl.ds(base, GW)], idxv)
    for q in range(GW // 80):
        pltpu.async_copy(
            src_hbm.at[idxv.at[pl.ds(q * 80, 80)]],
            rows.at[pl.ds(q * 80, 80), :], sem).wait()
    pltpu.sync_copy(rows, out_hbm.at[pl.ds(base, GW), :])


def kernel(x, edge_index, W0, b0, W1, b1, W2, b2, p_pool):
    f32 = jnp.float32
    dst = edge_index[1]
    src = edge_index[0]

    bd = _k1_build(dst, src).reshape(NPAD, NPAD)
    bb = _k15_cast(bd)
    cb = _k2_square(bb)
    dcf = _k3_dc(cb)

    n = NV
    di = jnp.arange(n)
    eye = di[:, None] == di[None, :]
    ah = jnp.where(eye, 2.0, cb[:n, :n].astype(f32))
    deg = ah.sum(axis=1)
    dinv = jnp.where(deg > 0, deg ** -0.5, 0.0)
    an = dinv[:, None] * ah * dinv[None, :]
    h = jax.nn.relu(an @ (x @ W0) + b0)
    score = jnp.tanh((h @ p_pool) / jnp.linalg.norm(p_pool))
    vals, perm = jax.lax.top_k(score, K)

    perm_pad = jnp.concatenate(
        [perm, jnp.full((KPAD - K,), NPAD - 1, jnp.int32)])
    vals_pad = jnp.concatenate([vals, jnp.zeros((KPAD - K,), f32)])
    s_ind, vfull = _k55_scatter(perm_pad, vals_pad)

    z0 = jnp.concatenate([s_ind[:, None], jnp.zeros((NPAD, D - 1), f32)], 1)
    w1v = _capp(cb, z0)
    w2v = _capp(cb, w1v)[:, 0]
    deg2f = w2v - dcf + 2.0
    dinv2f = jnp.where(s_ind > 0, deg2f ** -0.5, 0.0)

    hpad = jnp.pad(h, ((0, NPAD - n), (0, 0)))
    z2 = _t3(hpad, vfull, dinv2f, W1)
    u2 = _capp(cb, _capp(cb, z2))
    z3 = _t4(u2, z2, dinv2f, dcf, s_ind, W2, b1)
    u4 = _capp(cb, _capp(cb, z3))
    o_full = _t5(u4, z3, dinv2f, dcf, b2)

    o_pad = _k7_gather(o_full, perm_pad)
    out = o_pad[:K]
    batch = jnp.zeros((K,), jnp.int32)
    return out, batch
